# baseline (speedup 1.0000x reference)
import functools

import jax
import jax.numpy as jnp
from jax import lax
from jax.experimental import pallas as pl
from jax.experimental.pallas import tpu as pltpu
from jax.experimental.pallas import tpu_sc as plsc

_NC, _NS = 2, 16
_NW = _NC * _NS
_CH = 128
_GPO = 10
_OUTER = _CH * _GPO

_EMBED_BLK = 2000
_NODE_BLK = 1000


def _f32dot(a, b):
    return jnp.dot(a, b, preferred_element_type=jnp.float32)


def _embed_body(f_ref, w1, b1, w2, b2, w3, b3, v_ref):
    h = jax.nn.gelu(_f32dot(f_ref[...], w1[...]) + b1[...])
    h = jax.nn.gelu(_f32dot(h, w2[...]) + b2[...])
    v_ref[...] = _f32dot(h, w3[...]) + b3[...]


def _edge_body(g1_ref, g23_ref, v_ref,
               w1_bd, kb1_4, kw2_bd, kb2_4, kw3_bd, kb3_4,
               ws, bs, dw1, db1, dw2, db2, dw3, db3, u_ref):
    b = v_ref.shape[0]
    g1 = g1_ref[...]
    h4 = jax.nn.gelu(_f32dot(g23_ref[...], w1_bd[...]) + kb1_4[...])
    h4 = jax.nn.gelu(_f32dot(h4, kw2_bd[...]) + kb2_4[...])
    kern4 = _f32dot(h4, kw3_bd[...]) + kb3_4[...]
    mult4 = kern4 * g1
    s1 = mult4.reshape(b, 4, 128).sum(axis=1)
    integral = (s1[:, 0:32] + s1[:, 32:64] + s1[:, 64:96] + s1[:, 96:128]) \
        * (1.0 / 16.0)
    vt = jax.nn.gelu(_f32dot(v_ref[...], ws[...]) + bs[...] + integral)
    hd = jax.nn.gelu(_f32dot(vt, dw1[...]) + db1[...])
    hd = jax.nn.gelu(_f32dot(hd, dw2[...]) + db2[...])
    u_ref[...] = _f32dot(hd, dw3[...]) + db3[...]


def _make_sc_gather(e, dv, dx):
    assert e % _OUTER == 0
    nout = e // _OUTER
    nfull = nout // _NW
    rem = nout % _NW
    mesh = plsc.VectorSubcoreMesh(core_axis_name="c", subcore_axis_name="s")

    @functools.partial(
        pl.kernel, mesh=mesh,
        out_type=[jax.ShapeDtypeStruct((e, dv), jnp.float32),
                  jax.ShapeDtypeStruct((2 * e, dx), jnp.float32)],
        scratch_types=[
            pltpu.VMEM((_OUTER,), jnp.int32),
            pltpu.VMEM((2 * _OUTER,), jnp.int32),
            pltpu.VMEM((_OUTER, dv), jnp.float32),
            pltpu.VMEM((2 * _OUTER, dx), jnp.float32),
            pltpu.SemaphoreType.DMA,
            pltpu.SemaphoreType.DMA,
        ],
        compiler_params=pltpu.CompilerParams(use_tc_tiling_on_sc=False),
    )
    def sc_gather(v_hbm, x_hbm, idx_hbm, idx2_hbm, g1_hbm, g23_hbm,
                  idx_v, idx2_v, vrows, xrows, sem_v, sem_x):
        c = lax.axis_index("c")
        s = lax.axis_index("s")
        wid = s * _NC + c
        count = nfull + jnp.where(wid < rem, 1, 0)
        start = nfull * wid + jnp.minimum(wid, rem)

        def body(i, carry):
            ch = start + i
            ebase = pl.multiple_of(ch * _OUTER, 128)
            e2base = pl.multiple_of(ch * 2 * _OUTER, 128)
            pltpu.sync_copy(idx_hbm.at[pl.ds(ebase, _OUTER)], idx_v)
            pltpu.sync_copy(idx2_hbm.at[pl.ds(e2base, 2 * _OUTER)], idx2_v)
            copies = []
            for j in range(_GPO):
                sl = pl.ds(j * _CH, _CH)
                copies.append(pltpu.async_copy(
                    v_hbm.at[idx_v.at[sl]], vrows.at[sl], sem_v))
            for j in range(2 * _GPO):
                sl = pl.ds(j * _CH, _CH)
                copies.append(pltpu.async_copy(
                    x_hbm.at[idx2_v.at[sl]], xrows.at[sl], sem_x))
            for cp in copies:
                cp.wait()
            pltpu.sync_copy(vrows, g1_hbm.at[pl.ds(ebase, _OUTER)])
            pltpu.sync_copy(xrows, g23_hbm.at[pl.ds(e2base, 2 * _OUTER)])
            return carry

        lax.fori_loop(0, count, body, 0)

    return sc_gather


def _whole(shape):
    return pl.BlockSpec(shape, lambda i: (0,) * len(shape))


def kernel(x, F, neighbors_index, neighbors_row_splits,
           embed_params, kernel_params, decoder_params, W_skip, b_skip):
    n, d_in = x.shape
    e = neighbors_index.shape[0]
    deg = e // n
    d_f = F.shape[1]
    (ew1, eb1), (ew2, eb2), (ew3, eb3) = embed_params
    (kw1, kb1), (kw2, kb2), (kw3, kb3) = kernel_params
    (dw1, db1), (dw2, db2), (dw3, db3) = decoder_params
    h = ew1.shape[1]
    d_emb = ew3.shape[1]
    d_out = dw3.shape[1]

    r = _EMBED_BLK
    v = pl.pallas_call(
        _embed_body,
        grid=(n // r,),
        in_specs=[
            pl.BlockSpec((r, d_f), lambda i: (i, 0)),
            _whole(ew1.shape), _whole((1, h)),
            _whole(ew2.shape), _whole((1, h)),
            _whole(ew3.shape), _whole((1, d_emb)),
        ],
        out_specs=pl.BlockSpec((r, d_emb), lambda i: (i, 0)),
        out_shape=jax.ShapeDtypeStruct((n, d_emb), jnp.float32),
        compiler_params=pltpu.CompilerParams(
            dimension_semantics=("parallel",)),
    )(F, ew1, eb1.reshape(1, h), ew2, eb2.reshape(1, h),
      ew3, eb3.reshape(1, d_emb))

    dx = 16
    b = _NODE_BLK
    nblk = n // b
    x_pad = jnp.concatenate(
        [x, jnp.zeros((n, dx - d_in), jnp.float32)], axis=1)
    seg = jnp.arange(e, dtype=jnp.int32) // deg
    idx2 = jnp.stack([neighbors_index, seg], axis=1).reshape(2 * e)
    g1, g23 = _make_sc_gather(e, d_emb, dx)(
        v, x_pad, neighbors_index, idx2)
    g1 = g1.reshape(e // 4, 4 * d_emb)
    g23 = g23.reshape(e // 4, 8 * dx)

    w1a = jnp.concatenate([kw1[:d_in], jnp.zeros((dx - d_in, h), jnp.float32)])
    w1b = jnp.concatenate([kw1[d_in:], jnp.zeros((dx - d_in, h), jnp.float32)])
    bd = jax.scipy.linalg.block_diag
    w1ab = jnp.concatenate([w1a, w1b], axis=0)
    w1_bd = bd(w1ab, w1ab, w1ab, w1ab)
    kw2_bd = bd(kw2, kw2, kw2, kw2)
    kw3_bd = bd(kw3, kw3, kw3, kw3)
    kb1_4 = jnp.tile(kb1.reshape(1, h), (1, 4))
    kb2_4 = jnp.tile(kb2.reshape(1, h), (1, 4))
    kb3_4 = jnp.tile(kb3.reshape(1, d_emb), (1, 4))
    u = pl.pallas_call(
        _edge_body,
        grid=(nblk,),
        in_specs=[
            pl.BlockSpec((deg * b // 4, 4 * d_emb), lambda i: (i, 0)),
            pl.BlockSpec((deg * b // 4, 8 * dx), lambda i: (i, 0)),
            pl.BlockSpec((b, d_emb), lambda i: (i, 0)),
            _whole(w1_bd.shape), _whole((1, 4 * h)),
            _whole(kw2_bd.shape), _whole((1, 4 * h)),
            _whole(kw3_bd.shape), _whole((1, 4 * d_emb)),
            _whole(W_skip.shape), _whole((1, d_emb)),
            _whole(dw1.shape), _whole((1, h)),
            _whole(dw2.shape), _whole((1, h)),
            _whole(dw3.shape), _whole((1, d_out)),
        ],
        out_specs=pl.BlockSpec((b, d_out), lambda i: (i, 0)),
        out_shape=jax.ShapeDtypeStruct((n, d_out), jnp.float32),
        compiler_params=pltpu.CompilerParams(
            dimension_semantics=("parallel",)),
    )(g1, g23, v,
      w1_bd, kb1_4, kw2_bd, kb2_4,
      kw3_bd, kb3_4,
      W_skip, b_skip.reshape(1, d_emb),
      dw1, db1.reshape(1, h), dw2, db2.reshape(1, h),
      dw3, db3.reshape(1, d_out))
    return u


# speedup vs baseline: 2.1742x; 2.1742x over previous
import functools

import jax
import jax.numpy as jnp
from jax import lax
from jax.experimental import pallas as pl
from jax.experimental.pallas import tpu as pltpu
from jax.experimental.pallas import tpu_sc as plsc

_NC, _NS = 2, 16
_NW = _NC * _NS
_CH = 128
_GPO = 10
_OUTER = _CH * _GPO

_EMBED_BLK = 2000
_NODE_BLK = 1000


def _f32dot(a, b):
    return jnp.dot(a, b, preferred_element_type=jnp.float32)


def _embed_body(f_ref, w1, b1, w2, b2, w3, b3, v_ref):
    h = jax.nn.gelu(_f32dot(f_ref[...], w1[...]) + b1[...])
    h = jax.nn.gelu(_f32dot(h, w2[...]) + b2[...])
    v_ref[...] = _f32dot(h, w3[...]) + b3[...]


def _edge_body(g1_ref, g2_ref, v_ref, xp_ref,
               w1a_bd, w1b, kb1, kw2_bd, kb2_4, kw3_bd, kb3_4,
               ws, bs, dw1, db1, dw2, db2, dw3, db3, u_ref):
    b = v_ref.shape[0]
    r4 = g1_ref.shape[0]
    deg = 4 * r4 // b
    p = deg // 4
    a4 = _f32dot(g2_ref[...], w1a_bd[...])
    q = _f32dot(xp_ref[...], w1b[...]) + kb1[...]
    q4 = jnp.concatenate([q, q, q, q], axis=1)
    h4 = jax.nn.gelu(a4.reshape(b, p, 256) + q4.reshape(b, 1, 256)).reshape(r4, 256)
    h4 = jax.nn.gelu(_f32dot(h4, kw2_bd[...]) + kb2_4[...])
    kern4 = _f32dot(h4, kw3_bd[...]) + kb3_4[...]
    mult4 = kern4 * g1_ref[...]
    s1 = mult4.reshape(b, p, 128).sum(axis=1)
    integral = (s1[:, 0:32] + s1[:, 32:64] + s1[:, 64:96] + s1[:, 96:128]) \
        * (1.0 / deg)
    vt = jax.nn.gelu(_f32dot(v_ref[...], ws[...]) + bs[...] + integral)
    hd = jax.nn.gelu(_f32dot(vt, dw1[...]) + db1[...])
    hd = jax.nn.gelu(_f32dot(hd, dw2[...]) + db2[...])
    u_ref[...] = _f32dot(hd, dw3[...]) + db3[...]


def _make_sc_gather(n, e, dv, dx):
    assert e % _OUTER == 0
    nout = e // _OUTER
    nfull = nout // _NW
    rem = nout % _NW
    mesh = plsc.VectorSubcoreMesh(core_axis_name="c", subcore_axis_name="s")

    @functools.partial(
        pl.kernel, mesh=mesh,
        out_type=[jax.ShapeDtypeStruct((e, dv), jnp.float32),
                  jax.ShapeDtypeStruct((e, dx), jnp.float32)],
        scratch_types=[
            pltpu.VMEM((_OUTER,), jnp.int32),
            pltpu.VMEM((_OUTER, dv), jnp.float32),
            pltpu.VMEM((_OUTER, dx), jnp.float32),
            pltpu.SemaphoreType.DMA,
            pltpu.SemaphoreType.DMA,
        ],
        compiler_params=pltpu.CompilerParams(use_tc_tiling_on_sc=False),
    )
    def sc_gather(v_hbm, x_hbm, idx_hbm, g1_hbm, g2_hbm,
                  idx_v, vrows, xrows, sem_v, sem_x):
        c = lax.axis_index("c")
        s = lax.axis_index("s")
        wid = s * _NC + c
        count = nfull + jnp.where(wid < rem, 1, 0)
        start = nfull * wid + jnp.minimum(wid, rem)

        def body(i, carry):
            ch = start + i
            ebase = pl.multiple_of(ch * _OUTER, 128)
            pltpu.sync_copy(idx_hbm.at[pl.ds(ebase, _OUTER)], idx_v)
            copies = []
            for j in range(_GPO):
                sl = pl.ds(j * _CH, _CH)
                copies.append(pltpu.async_copy(
                    v_hbm.at[idx_v.at[sl]], vrows.at[sl], sem_v))
                copies.append(pltpu.async_copy(
                    x_hbm.at[idx_v.at[sl]], xrows.at[sl], sem_x))
            for cp in copies:
                cp.wait()
            pltpu.sync_copy(vrows, g1_hbm.at[pl.ds(ebase, _OUTER)])
            pltpu.sync_copy(xrows, g2_hbm.at[pl.ds(ebase, _OUTER)])
            return carry

        lax.fori_loop(0, count, body, 0)

    return sc_gather


def _whole(shape):
    return pl.BlockSpec(shape, lambda i: (0,) * len(shape))


def kernel(x, F, neighbors_index, neighbors_row_splits,
           embed_params, kernel_params, decoder_params, W_skip, b_skip):
    n, d_in = x.shape
    e = neighbors_index.shape[0]
    deg = e // n
    d_f = F.shape[1]
    (ew1, eb1), (ew2, eb2), (ew3, eb3) = embed_params
    (kw1, kb1), (kw2, kb2), (kw3, kb3) = kernel_params
    (dw1, db1), (dw2, db2), (dw3, db3) = decoder_params
    h = ew1.shape[1]
    d_emb = ew3.shape[1]
    d_out = dw3.shape[1]

    r = _EMBED_BLK
    v = pl.pallas_call(
        _embed_body,
        grid=(n // r,),
        in_specs=[
            pl.BlockSpec((r, d_f), lambda i: (i, 0)),
            _whole(ew1.shape), _whole((1, h)),
            _whole(ew2.shape), _whole((1, h)),
            _whole(ew3.shape), _whole((1, d_emb)),
        ],
        out_specs=pl.BlockSpec((r, d_emb), lambda i: (i, 0)),
        out_shape=jax.ShapeDtypeStruct((n, d_emb), jnp.float32),
        compiler_params=pltpu.CompilerParams(
            dimension_semantics=("parallel",)),
    )(F, ew1, eb1.reshape(1, h), ew2, eb2.reshape(1, h),
      ew3, eb3.reshape(1, d_emb))

    dx = 16
    x_pad = jnp.concatenate(
        [x, jnp.zeros((n, dx - d_in), jnp.float32)], axis=1)
    g1, g2 = _make_sc_gather(n, e, d_emb, dx)(v, x_pad, neighbors_index)

    w1a = jnp.concatenate([kw1[:d_in], jnp.zeros((dx - d_in, h), jnp.float32)])
    w1b = jnp.concatenate([kw1[d_in:], jnp.zeros((dx - d_in, h), jnp.float32)])
    bd = jax.scipy.linalg.block_diag
    w1a_bd = bd(w1a, w1a, w1a, w1a)
    kw2_bd = bd(kw2, kw2, kw2, kw2)
    kw3_bd = bd(kw3, kw3, kw3, kw3)
    kb2_4 = jnp.tile(kb2.reshape(1, h), (1, 4))
    kb3_4 = jnp.tile(kb3.reshape(1, d_emb), (1, 4))
    g1p = g1.reshape(e // 4, 4 * d_emb)
    g2p = g2.reshape(e // 4, 4 * dx)
    b = _NODE_BLK
    eb = b * deg
    u = pl.pallas_call(
        _edge_body,
        grid=(n // b,),
        in_specs=[
            pl.BlockSpec((eb // 4, 4 * d_emb), lambda i: (i, 0)),
            pl.BlockSpec((eb // 4, 4 * dx), lambda i: (i, 0)),
            pl.BlockSpec((b, d_emb), lambda i: (i, 0)),
            pl.BlockSpec((b, dx), lambda i: (i, 0)),
            _whole(w1a_bd.shape), _whole((dx, h)), _whole((1, h)),
            _whole(kw2_bd.shape), _whole((1, 4 * h)),
            _whole(kw3_bd.shape), _whole((1, 4 * d_emb)),
            _whole(W_skip.shape), _whole((1, d_emb)),
            _whole(dw1.shape), _whole((1, h)),
            _whole(dw2.shape), _whole((1, h)),
            _whole(dw3.shape), _whole((1, d_out)),
        ],
        out_specs=pl.BlockSpec((b, d_out), lambda i: (i, 0)),
        out_shape=jax.ShapeDtypeStruct((n, d_out), jnp.float32),
        compiler_params=pltpu.CompilerParams(
            dimension_semantics=("parallel",)),
    )(g1p, g2p, v, x_pad,
      w1a_bd, w1b, kb1.reshape(1, h), kw2_bd, kb2_4,
      kw3_bd, kb3_4,
      W_skip, b_skip.reshape(1, d_emb),
      dw1, db1.reshape(1, h), dw2, db2.reshape(1, h),
      dw3, db3.reshape(1, d_out))
    return u
